# baseline (speedup 1.0000x reference)
import jax
import jax.numpy as jnp
from jax.experimental import pallas as pl
from jax.experimental.pallas import tpu as pltpu

_B, _S = 8, 4
_T = _B * _S
_HIDDEN = 512
_INTER = 1024
_E = 8
_K = 2
_HH = _HIDDEN // 2
_NB13 = 3
_NB2 = 2
_NSTEP = 2 * _E


def _moe_kernel(x_ref, rw_ref, w13_hbm, w2_hbm, out_ref,
                buf13, buf2, sem13, sem2):
    x = x_ref[...]

    def cp13(s):
        e, h = divmod(s, 2)
        return pltpu.make_async_copy(
            w13_hbm.at[pl.ds(e, 1), pl.ds(h * _HH, _HH), :],
            buf13.at[pl.ds(s % _NB13, 1)],
            sem13.at[s % _NB13])

    def cp2(e):
        return pltpu.make_async_copy(
            w2_hbm.at[pl.ds(e, 1)],
            buf2.at[pl.ds(e % _NB2, 1)],
            sem2.at[e % _NB2])

    cp13(0).start()
    cp2(0).start()
    cp13(1).start()
    cp2(1).start()

    logits = jnp.dot(x, rw_ref[...].T,
                     preferred_element_type=jnp.float32)
    m = jnp.max(logits, axis=-1, keepdims=True)
    ex = jnp.exp(logits - m)
    probs = ex / jnp.sum(ex, axis=-1, keepdims=True)

    cols = jax.lax.broadcasted_iota(jnp.int32, probs.shape, 1)
    i1 = jnp.argmax(probs, axis=-1, keepdims=True)
    v1 = jnp.max(probs, axis=-1)
    masked = jnp.where(cols == i1, -1.0, probs)
    i2 = jnp.argmax(masked, axis=-1, keepdims=True)
    v2 = jnp.max(masked, axis=-1)
    denom = v1 + v2 + 1e-6
    sel = (cols == i1) | (cols == i2)
    dense_w = jnp.where(sel, probs, 0.0) / denom[:, None]

    acc = jnp.zeros((_T, _HIDDEN), jnp.float32)
    partial = None
    for s in range(_NSTEP):
        e, h = divmod(s, 2)
        cp13(s).wait()
        if s + 2 < _NSTEP:
            cp13(s + 2).start()
        xp = x[:, h * _HH:(h + 1) * _HH]
        p = jnp.dot(xp, buf13[s % _NB13],
                    preferred_element_type=jnp.float32)
        if h == 0:
            partial = p
        else:
            h13 = partial + p
            gate = h13[:, :_INTER]
            up = h13[:, _INTER:]
            hact = (gate * jax.nn.sigmoid(gate)) * up
            cp2(e).wait()
            out_e = jnp.dot(hact, buf2[e % _NB2],
                            preferred_element_type=jnp.float32)
            w_e = jnp.sum(jnp.where(cols == e, dense_w, 0.0), axis=-1)
            acc = acc + out_e * w_e[:, None]
            if e + 2 < _E:
                cp2(e + 2).start()

    out_ref[...] = acc


@jax.jit
def kernel(x, router_w, w13, w2):
    xt = x.reshape(_T, _HIDDEN)
    out = pl.pallas_call(
        _moe_kernel,
        in_specs=[
            pl.BlockSpec((_T, _HIDDEN), lambda: (0, 0)),
            pl.BlockSpec((_E, _HIDDEN), lambda: (0, 0)),
            pl.BlockSpec(memory_space=pl.ANY),
            pl.BlockSpec(memory_space=pl.ANY),
        ],
        out_specs=pl.BlockSpec((_T, _HIDDEN), lambda: (0, 0)),
        out_shape=jax.ShapeDtypeStruct((_T, _HIDDEN), jnp.float32),
        scratch_shapes=[
            pltpu.VMEM((_NB13, _HH, 2 * _INTER), jnp.float32),
            pltpu.VMEM((_NB2, _INTER, _HIDDEN), jnp.float32),
            pltpu.SemaphoreType.DMA((_NB13,)),
            pltpu.SemaphoreType.DMA((_NB2,)),
        ],
    )(xt, router_w, w13, w2)
    return out.reshape(_B, _S, _HIDDEN)
